# baseline (speedup 1.0000x reference)
import functools

import jax
import jax.numpy as jnp
from jax import lax
from jax.experimental import pallas as pl
from jax.experimental.pallas import tpu as pltpu
from jax.experimental.pallas import tpu_sc as plsc

N = 10000
D = 128
NC = 2
NS = 16
NW = NC * NS
B = 128
CH = 48
RPT = 640
ACC_ROWS = NS * RPT
CPT = 632
RB = 1000
GRID = N // RB

_mesh = plsc.VectorSubcoreMesh(core_axis_name="c", subcore_axis_name="s")


def _pb(num_edges):
    pb = -(-num_edges // (NW * B))
    return pb + (pb % 2)


def _make_deg_kernel(pbs):
    @functools.partial(
        pl.kernel,
        mesh=_mesh,
        out_type=jax.ShapeDtypeStruct((NC, ACC_ROWS), jnp.float32),
        scratch_types=[
            pltpu.VMEM((pbs, B), jnp.int32),
            pltpu.VMEM((pbs, B), jnp.float32),
            pltpu.VMEM_SHARED((ACC_ROWS,), jnp.float32),
        ],
    )
    def deg_kernel(dst_hbm, ones_hbm, zeros_hbm, out_hbm, dst_v, ones_v, acc):
        c = lax.axis_index("c")
        s = lax.axis_index("s")
        w = c * NS + s
        pltpu.sync_copy(dst_hbm.at[w], dst_v)
        pltpu.sync_copy(ones_hbm, ones_v)
        pltpu.sync_copy(zeros_hbm.at[pl.ds(s * RPT, RPT)],
                        acc.at[pl.ds(s * RPT, RPT)])
        plsc.subcore_barrier()

        def step(j, carry):
            pltpu.sync_copy(ones_v.at[j], acc.at[dst_v.at[j]], add=True)
            return carry

        lax.fori_loop(0, pbs, step, 0)
        plsc.subcore_barrier()
        pltpu.sync_copy(acc.at[pl.ds(s * RPT, RPT)],
                        out_hbm.at[c, pl.ds(s * RPT, RPT)])

    return deg_kernel


def _make_agg_kernel(pb):
    @functools.partial(
        pl.kernel,
        mesh=_mesh,
        out_type=jax.ShapeDtypeStruct((NC, N, D), jnp.float32),
        scratch_types=[
            pltpu.VMEM((CH, B), jnp.int32),
            pltpu.VMEM((CH, B), jnp.int32),
            pltpu.VMEM((B, D), jnp.float32),
            pltpu.VMEM((B, D), jnp.float32),
            pltpu.VMEM_SHARED((ACC_ROWS, D), jnp.float32),
            pltpu.SemaphoreType.DMA,
            pltpu.SemaphoreType.DMA,
            pltpu.SemaphoreType.DMA,
            pltpu.SemaphoreType.DMA,
            pltpu.SemaphoreType.DMA,
        ],
    )
    def agg_kernel(hs_hbm, src_hbm, dst_hbm, zeros_hbm, out_hbm,
                   src_v, dst_v, rows0, rows1, acc,
                   semg0, semg1, sems0, sems1, semz):
        c = lax.axis_index("c")
        s = lax.axis_index("s")
        w = c * NS + s
        pltpu.async_copy(zeros_hbm, acc.at[pl.ds(s * RPT, RPT)], semz)
        pltpu.sync_copy(src_hbm.at[w, pl.ds(0, CH)], src_v)
        pltpu.sync_copy(dst_hbm.at[w, pl.ds(0, CH)], dst_v)
        pltpu.async_copy(hs_hbm.at[src_v.at[0]], rows0, semg0)
        pltpu.make_async_copy(zeros_hbm,
                              acc.at[pl.ds(s * RPT, RPT)], semz).wait()
        plsc.subcore_barrier()

        rows = (rows0, rows1)
        semg = (semg0, semg1)
        sems = (sems0, sems1)

        def step(i, carry):
            for b in range(2):
                j = 2 * i + b
                jl = j % CH

                @pl.when(j > 0)
                def _():
                    pltpu.make_async_copy(rows[1 - b],
                                          acc.at[dst_v.at[(j - 1) % CH]],
                                          sems[1 - b]).wait()

                @pl.when(jnp.logical_and(jl == 0, j > 0))
                def _():
                    k0 = (j // CH) * CH
                    pltpu.sync_copy(src_hbm.at[w, pl.ds(k0, CH)], src_v)
                    pltpu.sync_copy(dst_hbm.at[w, pl.ds(k0, CH)], dst_v)
                    pltpu.async_copy(hs_hbm.at[src_v.at[0]],
                                     rows[b], semg[b])

                @pl.when(jnp.logical_and(j + 1 < pb, jl + 1 < CH))
                def _():
                    pltpu.async_copy(hs_hbm.at[src_v.at[jl + 1]],
                                     rows[1 - b], semg[1 - b])

                pltpu.make_async_copy(hs_hbm.at[src_v.at[jl]],
                                      rows[b], semg[b]).wait()
                pltpu.async_copy(rows[b], acc.at[dst_v.at[jl]],
                                 sems[b], add=True)
            return carry

        lax.fori_loop(0, pb // 2, step, 0)
        pltpu.make_async_copy(rows[1], acc.at[dst_v.at[(pb - 1) % CH]],
                              sems[1]).wait()
        plsc.subcore_barrier()
        r0 = jnp.minimum(s * CPT, N - CPT)
        pltpu.sync_copy(acc.at[pl.ds(r0, CPT)],
                        out_hbm.at[c, pl.ds(r0, CPT)])

    return agg_kernel


def _pre1_body(x_ref, w_ref, d0_ref, d1_ref, hs_ref, dis_ref):
    dis = lax.rsqrt(d0_ref[...] + d1_ref[...])
    h = jnp.dot(x_ref[...], w_ref[...], preferred_element_type=jnp.float32)
    hs_ref[...] = h * dis
    dis_ref[...] = dis


def _pre1(x, W1, d0, d1):
    return pl.pallas_call(
        _pre1_body,
        grid=(GRID,),
        in_specs=[
            pl.BlockSpec((RB, D), lambda i: (i, 0)),
            pl.BlockSpec((D, D), lambda i: (0, 0)),
            pl.BlockSpec((RB, 1), lambda i: (i, 0)),
            pl.BlockSpec((RB, 1), lambda i: (i, 0)),
        ],
        out_specs=[
            pl.BlockSpec((RB, D), lambda i: (i, 0)),
            pl.BlockSpec((RB, 1), lambda i: (i, 0)),
        ],
        out_shape=[
            jax.ShapeDtypeStruct((N, D), jnp.float32),
            jax.ShapeDtypeStruct((N, 1), jnp.float32),
        ],
    )(x, W1, d0, d1)


def _mid_body(a0_ref, a1_ref, dis_ref, b_ref, w_ref, hs_ref):
    dis = dis_ref[...]
    t = dis * (a0_ref[0] + a1_ref[0]) + b_ref[...]
    t = jnp.where(t > 0.0, t, jnp.exp(t) - 1.0)
    hs_ref[...] = jnp.dot(
        t, w_ref[...], preferred_element_type=jnp.float32) * dis


def _mid(a, dis, bias, W):
    return pl.pallas_call(
        _mid_body,
        grid=(GRID,),
        in_specs=[
            pl.BlockSpec((1, RB, D), lambda i: (0, i, 0)),
            pl.BlockSpec((1, RB, D), lambda i: (1, i, 0)),
            pl.BlockSpec((RB, 1), lambda i: (i, 0)),
            pl.BlockSpec((1, D), lambda i: (0, 0)),
            pl.BlockSpec((D, D), lambda i: (0, 0)),
        ],
        out_specs=pl.BlockSpec((RB, D), lambda i: (i, 0)),
        out_shape=jax.ShapeDtypeStruct((N, D), jnp.float32),
    )(a, a, dis, bias, W)


def _fin_body(a0_ref, a1_ref, dis_ref, b_ref, o_ref):
    o_ref[...] = dis_ref[...] * (a0_ref[0] + a1_ref[0]) + b_ref[...]


def _fin(a, dis, bias):
    return pl.pallas_call(
        _fin_body,
        grid=(GRID,),
        in_specs=[
            pl.BlockSpec((1, RB, D), lambda i: (0, i, 0)),
            pl.BlockSpec((1, RB, D), lambda i: (1, i, 0)),
            pl.BlockSpec((RB, 1), lambda i: (i, 0)),
            pl.BlockSpec((1, D), lambda i: (0, 0)),
        ],
        out_specs=pl.BlockSpec((RB, D), lambda i: (i, 0)),
        out_shape=jax.ShapeDtypeStruct((N, D), jnp.float32),
    )(a, a, dis, bias)


def kernel(x, edge_index, W1, b1, W2, b2, W3, b3):
    e = edge_index.shape[1]
    e2 = e + N
    pb = _pb(e2)
    epad = NW * B * pb
    npad = epad - e2

    loop = lax.iota(jnp.int32, N)
    pad_ar = lax.iota(jnp.int32, npad)
    srcp = jnp.concatenate([edge_index[0], loop, pad_ar % N])
    dstp = jnp.concatenate([edge_index[1], loop, N + (pad_ar % 16)])
    pbs = -(-pb // CH) * CH
    src3 = jnp.pad(srcp.reshape(NW, pb, B), ((0, 0), (0, pbs - pb), (0, 0)))
    dpad = N + (lax.iota(jnp.int32, B) % 16)
    dst3 = jnp.concatenate(
        [dstp.reshape(NW, pb, B),
         jnp.broadcast_to(dpad, (NW, pbs - pb, B))], axis=1)

    zcol = jnp.zeros((ACC_ROWS,), jnp.float32)
    zrows = jnp.zeros((RPT, D), jnp.float32)
    ones2 = jnp.ones((pbs, B), jnp.float32)

    degp = _make_deg_kernel(pbs)(dst3, ones2, zcol)
    d0 = degp[0, :N, None]
    d1 = degp[1, :N, None]

    agg = _make_agg_kernel(pb)
    hs1, dis = _pre1(x, W1, d0, d1)
    a1 = agg(hs1, src3, dst3, zrows)
    hs2 = _mid(a1, dis, b1.reshape(1, D), W2)
    a2 = agg(hs2, src3, dst3, zrows)
    hs3 = _mid(a2, dis, b2.reshape(1, D), W3)
    a3 = agg(hs3, src3, dst3, zrows)
    return _fin(a3, dis, b3.reshape(1, D))


# speedup vs baseline: 1.0379x; 1.0379x over previous
import functools

import jax
import jax.numpy as jnp
from jax import lax
from jax.experimental import pallas as pl
from jax.experimental.pallas import tpu as pltpu
from jax.experimental.pallas import tpu_sc as plsc

N = 10000
D = 128
NC = 2
NS = 16
NW = NC * NS
B = 128
CH = 48
RPT = 640
ACC_ROWS = NS * RPT
CPT = 632
RB = 1000
GRID = N // RB

_mesh = plsc.VectorSubcoreMesh(core_axis_name="c", subcore_axis_name="s")


def _pb(num_edges):
    pb = -(-num_edges // (NW * B))
    return pb + (pb % 2)


def _make_deg_kernel(pb, pbs):
    @functools.partial(
        pl.kernel,
        mesh=_mesh,
        out_type=jax.ShapeDtypeStruct((NC, ACC_ROWS), jnp.float32),
        scratch_types=[
            pltpu.VMEM((pbs, B), jnp.int32),
            pltpu.VMEM((pbs, B), jnp.float32),
            pltpu.VMEM_SHARED((ACC_ROWS,), jnp.float32),
        ],
    )
    def deg_kernel(dst_hbm, ones_hbm, zeros_hbm, out_hbm, dst_v, ones_v, acc):
        c = lax.axis_index("c")
        s = lax.axis_index("s")
        w = c * NS + s
        pltpu.sync_copy(dst_hbm.at[w], dst_v)
        pltpu.sync_copy(ones_hbm, ones_v)
        pltpu.sync_copy(zeros_hbm.at[pl.ds(s * RPT, RPT)],
                        acc.at[pl.ds(s * RPT, RPT)])
        plsc.subcore_barrier()

        def step(j, carry):
            pltpu.sync_copy(ones_v.at[j], acc.at[dst_v.at[j]], add=True)
            return carry

        lax.fori_loop(0, pb, step, 0)
        plsc.subcore_barrier()
        pltpu.sync_copy(acc.at[pl.ds(s * RPT, RPT)],
                        out_hbm.at[c, pl.ds(s * RPT, RPT)])

    return deg_kernel


def _make_agg_kernel(pb):
    @functools.partial(
        pl.kernel,
        mesh=_mesh,
        out_type=jax.ShapeDtypeStruct((NC, N, D), jnp.float32),
        scratch_types=[
            pltpu.VMEM((CH, B), jnp.int32),
            pltpu.VMEM((CH, B), jnp.int32),
            pltpu.VMEM((B, D), jnp.float32),
            pltpu.VMEM((B, D), jnp.float32),
            pltpu.VMEM_SHARED((ACC_ROWS, D), jnp.float32),
            pltpu.SemaphoreType.DMA,
            pltpu.SemaphoreType.DMA,
            pltpu.SemaphoreType.DMA,
            pltpu.SemaphoreType.DMA,
            pltpu.SemaphoreType.DMA,
        ],
    )
    def agg_kernel(hs_hbm, src_hbm, dst_hbm, zeros_hbm, out_hbm,
                   src_v, dst_v, rows0, rows1, acc,
                   semg0, semg1, sems0, sems1, semz):
        c = lax.axis_index("c")
        s = lax.axis_index("s")
        w = c * NS + s
        pltpu.async_copy(zeros_hbm, acc.at[pl.ds(s * RPT, RPT)], semz)
        pltpu.sync_copy(src_hbm.at[w, pl.ds(0, CH)], src_v)
        pltpu.sync_copy(dst_hbm.at[w, pl.ds(0, CH)], dst_v)
        pltpu.async_copy(hs_hbm.at[src_v.at[0]], rows0, semg0)
        pltpu.make_async_copy(zeros_hbm,
                              acc.at[pl.ds(s * RPT, RPT)], semz).wait()
        plsc.subcore_barrier()

        rows = (rows0, rows1)
        semg = (semg0, semg1)
        sems = (sems0, sems1)

        def step(i, carry):
            for b in range(2):
                j = 2 * i + b
                jl = j % CH

                @pl.when(j > 0)
                def _():
                    pltpu.make_async_copy(rows[1 - b],
                                          acc.at[dst_v.at[(j - 1) % CH]],
                                          sems[1 - b]).wait()

                @pl.when(jnp.logical_and(jl == 0, j > 0))
                def _():
                    k0 = (j // CH) * CH
                    pltpu.sync_copy(src_hbm.at[w, pl.ds(k0, CH)], src_v)
                    pltpu.sync_copy(dst_hbm.at[w, pl.ds(k0, CH)], dst_v)
                    pltpu.async_copy(hs_hbm.at[src_v.at[0]],
                                     rows[b], semg[b])

                @pl.when(jnp.logical_and(j + 1 < pb, jl + 1 < CH))
                def _():
                    pltpu.async_copy(hs_hbm.at[src_v.at[jl + 1]],
                                     rows[1 - b], semg[1 - b])

                pltpu.make_async_copy(hs_hbm.at[src_v.at[jl]],
                                      rows[b], semg[b]).wait()
                pltpu.async_copy(rows[b], acc.at[dst_v.at[jl]],
                                 sems[b], add=True)
            return carry

        lax.fori_loop(0, pb // 2, step, 0)
        pltpu.make_async_copy(rows[1], acc.at[dst_v.at[(pb - 1) % CH]],
                              sems[1]).wait()
        plsc.subcore_barrier()
        r0 = jnp.minimum(s * CPT, N - CPT)
        pltpu.sync_copy(acc.at[pl.ds(r0, CPT)],
                        out_hbm.at[c, pl.ds(r0, CPT)])

    return agg_kernel


def _pre1_body(x_ref, w_ref, d0_ref, d1_ref, hs_ref, dis_ref):
    dis = lax.rsqrt(d0_ref[...] + d1_ref[...] + 1.0)
    h = jnp.dot(x_ref[...], w_ref[...], preferred_element_type=jnp.float32)
    hs_ref[...] = h * dis
    dis_ref[...] = dis


def _pre1(x, W1, d0, d1):
    return pl.pallas_call(
        _pre1_body,
        grid=(GRID,),
        in_specs=[
            pl.BlockSpec((RB, D), lambda i: (i, 0)),
            pl.BlockSpec((D, D), lambda i: (0, 0)),
            pl.BlockSpec((RB, 1), lambda i: (i, 0)),
            pl.BlockSpec((RB, 1), lambda i: (i, 0)),
        ],
        out_specs=[
            pl.BlockSpec((RB, D), lambda i: (i, 0)),
            pl.BlockSpec((RB, 1), lambda i: (i, 0)),
        ],
        out_shape=[
            jax.ShapeDtypeStruct((N, D), jnp.float32),
            jax.ShapeDtypeStruct((N, 1), jnp.float32),
        ],
    )(x, W1, d0, d1)


def _mid_body(a0_ref, a1_ref, hp_ref, dis_ref, b_ref, w_ref, hs_ref):
    dis = dis_ref[...]
    t = dis * (a0_ref[0] + a1_ref[0] + hp_ref[...]) + b_ref[...]
    t = jnp.where(t > 0.0, t, jnp.exp(t) - 1.0)
    hs_ref[...] = jnp.dot(
        t, w_ref[...], preferred_element_type=jnp.float32) * dis


def _mid(a, hs_prev, dis, bias, W):
    return pl.pallas_call(
        _mid_body,
        grid=(GRID,),
        in_specs=[
            pl.BlockSpec((1, RB, D), lambda i: (0, i, 0)),
            pl.BlockSpec((1, RB, D), lambda i: (1, i, 0)),
            pl.BlockSpec((RB, D), lambda i: (i, 0)),
            pl.BlockSpec((RB, 1), lambda i: (i, 0)),
            pl.BlockSpec((1, D), lambda i: (0, 0)),
            pl.BlockSpec((D, D), lambda i: (0, 0)),
        ],
        out_specs=pl.BlockSpec((RB, D), lambda i: (i, 0)),
        out_shape=jax.ShapeDtypeStruct((N, D), jnp.float32),
    )(a, a, hs_prev, dis, bias, W)


def _fin_body(a0_ref, a1_ref, hp_ref, dis_ref, b_ref, o_ref):
    o_ref[...] = (dis_ref[...] * (a0_ref[0] + a1_ref[0] + hp_ref[...])
                  + b_ref[...])


def _fin(a, hs_prev, dis, bias):
    return pl.pallas_call(
        _fin_body,
        grid=(GRID,),
        in_specs=[
            pl.BlockSpec((1, RB, D), lambda i: (0, i, 0)),
            pl.BlockSpec((1, RB, D), lambda i: (1, i, 0)),
            pl.BlockSpec((RB, D), lambda i: (i, 0)),
            pl.BlockSpec((RB, 1), lambda i: (i, 0)),
            pl.BlockSpec((1, D), lambda i: (0, 0)),
        ],
        out_specs=pl.BlockSpec((RB, D), lambda i: (i, 0)),
        out_shape=jax.ShapeDtypeStruct((N, D), jnp.float32),
    )(a, a, hs_prev, dis, bias)


def kernel(x, edge_index, W1, b1, W2, b2, W3, b3):
    e = edge_index.shape[1]
    pb = _pb(e)
    epad = NW * B * pb
    npad = epad - e

    pad_ar = lax.iota(jnp.int32, npad)
    srcp = jnp.concatenate([edge_index[0], pad_ar % N])
    dstp = jnp.concatenate([edge_index[1], N + (pad_ar % 16)])
    pbs = -(-pb // CH) * CH
    src3 = jnp.pad(srcp.reshape(NW, pb, B), ((0, 0), (0, pbs - pb), (0, 0)))
    dpad = N + (lax.iota(jnp.int32, B) % 16)
    dst3 = jnp.concatenate(
        [dstp.reshape(NW, pb, B),
         jnp.broadcast_to(dpad, (NW, pbs - pb, B))], axis=1)

    zcol = jnp.zeros((ACC_ROWS,), jnp.float32)
    zrows = jnp.zeros((RPT, D), jnp.float32)
    ones2 = jnp.ones((pbs, B), jnp.float32)

    degp = _make_deg_kernel(pb, pbs)(dst3, ones2, zcol)
    d0 = degp[0, :N, None]
    d1 = degp[1, :N, None]

    agg = _make_agg_kernel(pb)
    hs1, dis = _pre1(x, W1, d0, d1)
    a1 = agg(hs1, src3, dst3, zrows)
    hs2 = _mid(a1, hs1, dis, b1.reshape(1, D), W2)
    a2 = agg(hs2, src3, dst3, zrows)
    hs3 = _mid(a2, hs2, dis, b2.reshape(1, D), W3)
    a3 = agg(hs3, src3, dst3, zrows)
    return _fin(a3, hs3, dis, b3.reshape(1, D))


# speedup vs baseline: 1.0511x; 1.0127x over previous
import functools

import jax
import jax.numpy as jnp
from jax import lax
from jax.experimental import pallas as pl
from jax.experimental.pallas import tpu as pltpu
from jax.experimental.pallas import tpu_sc as plsc

N = 10000
D = 128
NC = 2
NS = 16
NW = NC * NS
B = 128
CH = 48
RPT = 640
ACC_ROWS = NS * RPT
CPT = 632
RB = 1000
GRID = N // RB

_mesh = plsc.VectorSubcoreMesh(core_axis_name="c", subcore_axis_name="s")


def _pb(num_edges):
    pb = -(-num_edges // (NW * B))
    return pb + (pb % 2)


def _make_deg_kernel(pb, pbs):
    @functools.partial(
        pl.kernel,
        mesh=_mesh,
        out_type=jax.ShapeDtypeStruct((NC, ACC_ROWS), jnp.float32),
        scratch_types=[
            pltpu.VMEM((pbs, B), jnp.int32),
            pltpu.VMEM((B,), jnp.float32),
            pltpu.VMEM_SHARED((ACC_ROWS,), jnp.float32),
        ],
    )
    def deg_kernel(dst_hbm, zeros_hbm, out_hbm, dst_v, ones_v, acc):
        c = lax.axis_index("c")
        s = lax.axis_index("s")
        w = c * NS + s
        pltpu.sync_copy(dst_hbm.at[w], dst_v)
        pltpu.sync_copy(zeros_hbm.at[pl.ds(s * RPT, RPT)],
                        acc.at[pl.ds(s * RPT, RPT)])
        for k in range(B // 16):
            ones_v[pl.ds(16 * k, 16)] = jnp.ones((16,), jnp.float32)
        plsc.subcore_barrier()

        def step(j, carry):
            pltpu.sync_copy(ones_v, acc.at[dst_v.at[j]], add=True)
            return carry

        lax.fori_loop(0, pb, step, 0)
        plsc.subcore_barrier()
        pltpu.sync_copy(acc.at[pl.ds(s * RPT, RPT)],
                        out_hbm.at[c, pl.ds(s * RPT, RPT)])

    return deg_kernel


def _make_agg_kernel(pb):
    @functools.partial(
        pl.kernel,
        mesh=_mesh,
        out_type=jax.ShapeDtypeStruct((NC, N, D), jnp.float32),
        scratch_types=[
            pltpu.VMEM((CH, B), jnp.int32),
            pltpu.VMEM((CH, B), jnp.int32),
            pltpu.VMEM((B, D), jnp.float32),
            pltpu.VMEM((B, D), jnp.float32),
            pltpu.VMEM_SHARED((ACC_ROWS, D), jnp.float32),
            pltpu.SemaphoreType.DMA,
            pltpu.SemaphoreType.DMA,
            pltpu.SemaphoreType.DMA,
            pltpu.SemaphoreType.DMA,
            pltpu.SemaphoreType.DMA,
        ],
    )
    def agg_kernel(hs_hbm, src_hbm, dst_hbm, zeros_hbm, out_hbm,
                   src_v, dst_v, rows0, rows1, acc,
                   semg0, semg1, sems0, sems1, semz):
        c = lax.axis_index("c")
        s = lax.axis_index("s")
        w = c * NS + s
        pltpu.async_copy(zeros_hbm, acc.at[pl.ds(s * RPT, RPT)], semz)
        pltpu.sync_copy(src_hbm.at[w, pl.ds(0, CH)], src_v)
        pltpu.sync_copy(dst_hbm.at[w, pl.ds(0, CH)], dst_v)
        pltpu.async_copy(hs_hbm.at[src_v.at[0]], rows0, semg0)
        pltpu.make_async_copy(zeros_hbm,
                              acc.at[pl.ds(s * RPT, RPT)], semz).wait()
        plsc.subcore_barrier()

        rows = (rows0, rows1)
        semg = (semg0, semg1)
        sems = (sems0, sems1)

        def step(i, carry):
            for b in range(2):
                j = 2 * i + b
                jl = j % CH

                @pl.when(j > 0)
                def _():
                    pltpu.make_async_copy(rows[1 - b],
                                          acc.at[dst_v.at[(j - 1) % CH]],
                                          sems[1 - b]).wait()

                @pl.when(jnp.logical_and(jl == 0, j > 0))
                def _():
                    k0 = (j // CH) * CH
                    pltpu.sync_copy(src_hbm.at[w, pl.ds(k0, CH)], src_v)
                    pltpu.sync_copy(dst_hbm.at[w, pl.ds(k0, CH)], dst_v)
                    pltpu.async_copy(hs_hbm.at[src_v.at[0]],
                                     rows[b], semg[b])

                @pl.when(jnp.logical_and(j + 1 < pb, jl + 1 < CH))
                def _():
                    pltpu.async_copy(hs_hbm.at[src_v.at[jl + 1]],
                                     rows[1 - b], semg[1 - b])

                pltpu.make_async_copy(hs_hbm.at[src_v.at[jl]],
                                      rows[b], semg[b]).wait()
                pltpu.async_copy(rows[b], acc.at[dst_v.at[jl]],
                                 sems[b], add=True)
            return carry

        lax.fori_loop(0, pb // 2, step, 0)
        pltpu.make_async_copy(rows[1], acc.at[dst_v.at[(pb - 1) % CH]],
                              sems[1]).wait()
        plsc.subcore_barrier()
        r0 = jnp.minimum(s * CPT, N - CPT)
        pltpu.sync_copy(acc.at[pl.ds(r0, CPT)],
                        out_hbm.at[c, pl.ds(r0, CPT)])

    return agg_kernel


def _mm_body(x_ref, w_ref, h_ref):
    h_ref[...] = jnp.dot(x_ref[...], w_ref[...],
                         preferred_element_type=jnp.float32)


def _mm(x, W1):
    return pl.pallas_call(
        _mm_body,
        grid=(GRID,),
        in_specs=[
            pl.BlockSpec((RB, D), lambda i: (i, 0)),
            pl.BlockSpec((D, D), lambda i: (0, 0)),
        ],
        out_specs=pl.BlockSpec((RB, D), lambda i: (i, 0)),
        out_shape=jax.ShapeDtypeStruct((N, D), jnp.float32),
    )(x, W1)


def _scale_body(h_ref, d0_ref, d1_ref, hs_ref, dis_ref):
    dis = lax.rsqrt(d0_ref[...] + d1_ref[...] + 1.0)
    hs_ref[...] = h_ref[...] * dis
    dis_ref[...] = dis


def _scale(h, d0, d1):
    return pl.pallas_call(
        _scale_body,
        grid=(GRID,),
        in_specs=[
            pl.BlockSpec((RB, D), lambda i: (i, 0)),
            pl.BlockSpec((RB, 1), lambda i: (i, 0)),
            pl.BlockSpec((RB, 1), lambda i: (i, 0)),
        ],
        out_specs=[
            pl.BlockSpec((RB, D), lambda i: (i, 0)),
            pl.BlockSpec((RB, 1), lambda i: (i, 0)),
        ],
        out_shape=[
            jax.ShapeDtypeStruct((N, D), jnp.float32),
            jax.ShapeDtypeStruct((N, 1), jnp.float32),
        ],
    )(h, d0, d1)


def _mid_body(a0_ref, a1_ref, hp_ref, dis_ref, b_ref, w_ref, hs_ref):
    dis = dis_ref[...]
    t = dis * (a0_ref[0] + a1_ref[0] + hp_ref[...]) + b_ref[...]
    t = jnp.where(t > 0.0, t, jnp.exp(t) - 1.0)
    hs_ref[...] = jnp.dot(
        t, w_ref[...], preferred_element_type=jnp.float32) * dis


def _mid(a, hs_prev, dis, bias, W):
    return pl.pallas_call(
        _mid_body,
        grid=(GRID,),
        in_specs=[
            pl.BlockSpec((1, RB, D), lambda i: (0, i, 0)),
            pl.BlockSpec((1, RB, D), lambda i: (1, i, 0)),
            pl.BlockSpec((RB, D), lambda i: (i, 0)),
            pl.BlockSpec((RB, 1), lambda i: (i, 0)),
            pl.BlockSpec((1, D), lambda i: (0, 0)),
            pl.BlockSpec((D, D), lambda i: (0, 0)),
        ],
        out_specs=pl.BlockSpec((RB, D), lambda i: (i, 0)),
        out_shape=jax.ShapeDtypeStruct((N, D), jnp.float32),
    )(a, a, hs_prev, dis, bias, W)


def _fin_body(a0_ref, a1_ref, hp_ref, dis_ref, b_ref, o_ref):
    o_ref[...] = (dis_ref[...] * (a0_ref[0] + a1_ref[0] + hp_ref[...])
                  + b_ref[...])


def _fin(a, hs_prev, dis, bias):
    return pl.pallas_call(
        _fin_body,
        grid=(GRID,),
        in_specs=[
            pl.BlockSpec((1, RB, D), lambda i: (0, i, 0)),
            pl.BlockSpec((1, RB, D), lambda i: (1, i, 0)),
            pl.BlockSpec((RB, D), lambda i: (i, 0)),
            pl.BlockSpec((RB, 1), lambda i: (i, 0)),
            pl.BlockSpec((1, D), lambda i: (0, 0)),
        ],
        out_specs=pl.BlockSpec((RB, D), lambda i: (i, 0)),
        out_shape=jax.ShapeDtypeStruct((N, D), jnp.float32),
    )(a, a, hs_prev, dis, bias)


def kernel(x, edge_index, W1, b1, W2, b2, W3, b3):
    e = edge_index.shape[1]
    pb = _pb(e)
    epad = NW * B * pb
    npad = epad - e

    pad_ar = lax.iota(jnp.int32, npad)
    srcp = jnp.concatenate([edge_index[0], pad_ar % N])
    dstp = jnp.concatenate([edge_index[1], N + (pad_ar % 16)])
    pbs = -(-pb // CH) * CH
    src3 = jnp.pad(srcp.reshape(NW, pb, B), ((0, 0), (0, pbs - pb), (0, 0)))
    dpad = N + (lax.iota(jnp.int32, B) % 16)
    dst3 = jnp.concatenate(
        [dstp.reshape(NW, pb, B),
         jnp.broadcast_to(dpad, (NW, pbs - pb, B))], axis=1)

    zcol = jnp.zeros((ACC_ROWS,), jnp.float32)
    zrows = jnp.zeros((RPT, D), jnp.float32)

    h1 = _mm(x, W1)
    degp = _make_deg_kernel(pb, pbs)(dst3, zcol)
    d0 = degp[0, :N, None]
    d1 = degp[1, :N, None]

    agg = _make_agg_kernel(pb)
    hs1, dis = _scale(h1, d0, d1)
    a1 = agg(hs1, src3, dst3, zrows)
    hs2 = _mid(a1, hs1, dis, b1.reshape(1, D), W2)
    a2 = agg(hs2, src3, dst3, zrows)
    hs3 = _mid(a2, hs2, dis, b2.reshape(1, D), W3)
    a3 = agg(hs3, src3, dst3, zrows)
    return _fin(a3, hs3, dis, b3.reshape(1, D))


# speedup vs baseline: 1.0663x; 1.0145x over previous
import functools

import jax
import jax.numpy as jnp
from jax import lax
from jax.experimental import pallas as pl
from jax.experimental.pallas import tpu as pltpu
from jax.experimental.pallas import tpu_sc as plsc

N = 10000
D = 128
NC = 2
NS = 16
NW = NC * NS
B = 128
CH = 48
RPT = 640
ACC_ROWS = NS * RPT
CPT = 632
RB = 2000
GRID = N // RB

_mesh = plsc.VectorSubcoreMesh(core_axis_name="c", subcore_axis_name="s")


def _pb(num_edges):
    pb = -(-num_edges // (NW * B))
    return pb + (pb % 2)


def _make_deg_kernel(pb, pbs):
    @functools.partial(
        pl.kernel,
        mesh=_mesh,
        out_type=jax.ShapeDtypeStruct((NC, ACC_ROWS), jnp.float32),
        scratch_types=[
            pltpu.VMEM((pbs, B), jnp.int32),
            pltpu.VMEM((B,), jnp.float32),
            pltpu.VMEM_SHARED((ACC_ROWS,), jnp.float32),
        ],
    )
    def deg_kernel(dst_hbm, zeros_hbm, out_hbm, dst_v, ones_v, acc):
        c = lax.axis_index("c")
        s = lax.axis_index("s")
        w = c * NS + s
        pltpu.sync_copy(dst_hbm.at[w], dst_v)
        pltpu.sync_copy(zeros_hbm.at[pl.ds(s * RPT, RPT)],
                        acc.at[pl.ds(s * RPT, RPT)])
        for k in range(B // 16):
            ones_v[pl.ds(16 * k, 16)] = jnp.ones((16,), jnp.float32)
        plsc.subcore_barrier()

        def step(j, carry):
            pltpu.sync_copy(ones_v, acc.at[dst_v.at[j]], add=True)
            return carry

        lax.fori_loop(0, pb, step, 0)
        plsc.subcore_barrier()
        pltpu.sync_copy(acc.at[pl.ds(s * RPT, RPT)],
                        out_hbm.at[c, pl.ds(s * RPT, RPT)])

    return deg_kernel


def _make_agg_kernel(pb):
    @functools.partial(
        pl.kernel,
        mesh=_mesh,
        out_type=jax.ShapeDtypeStruct((NC, N, D), jnp.float32),
        scratch_types=[
            pltpu.VMEM((CH, B), jnp.int32),
            pltpu.VMEM((CH, B), jnp.int32),
            pltpu.VMEM((B, D), jnp.float32),
            pltpu.VMEM((B, D), jnp.float32),
            pltpu.VMEM_SHARED((ACC_ROWS, D), jnp.float32),
            pltpu.SemaphoreType.DMA,
            pltpu.SemaphoreType.DMA,
            pltpu.SemaphoreType.DMA,
            pltpu.SemaphoreType.DMA,
            pltpu.SemaphoreType.DMA,
        ],
    )
    def agg_kernel(hs_hbm, src_hbm, dst_hbm, zeros_hbm, out_hbm,
                   src_v, dst_v, rows0, rows1, acc,
                   semg0, semg1, sems0, sems1, semz):
        c = lax.axis_index("c")
        s = lax.axis_index("s")
        w = c * NS + s
        pltpu.async_copy(zeros_hbm, acc.at[pl.ds(s * RPT, RPT)], semz)
        pltpu.sync_copy(src_hbm.at[w, pl.ds(0, CH)], src_v)
        pltpu.sync_copy(dst_hbm.at[w, pl.ds(0, CH)], dst_v)
        pltpu.async_copy(hs_hbm.at[src_v.at[0]], rows0, semg0)
        pltpu.make_async_copy(zeros_hbm,
                              acc.at[pl.ds(s * RPT, RPT)], semz).wait()
        plsc.subcore_barrier()

        rows = (rows0, rows1)
        semg = (semg0, semg1)
        sems = (sems0, sems1)

        def step(i, carry):
            for b in range(2):
                j = 2 * i + b
                jl = j % CH

                @pl.when(j > 0)
                def _():
                    pltpu.make_async_copy(rows[1 - b],
                                          acc.at[dst_v.at[(j - 1) % CH]],
                                          sems[1 - b]).wait()

                @pl.when(jnp.logical_and(jl == 0, j > 0))
                def _():
                    k0 = (j // CH) * CH
                    pltpu.sync_copy(src_hbm.at[w, pl.ds(k0, CH)], src_v)
                    pltpu.sync_copy(dst_hbm.at[w, pl.ds(k0, CH)], dst_v)
                    pltpu.async_copy(hs_hbm.at[src_v.at[0]],
                                     rows[b], semg[b])

                @pl.when(jnp.logical_and(j + 1 < pb, jl + 1 < CH))
                def _():
                    pltpu.async_copy(hs_hbm.at[src_v.at[jl + 1]],
                                     rows[1 - b], semg[1 - b])

                pltpu.make_async_copy(hs_hbm.at[src_v.at[jl]],
                                      rows[b], semg[b]).wait()
                pltpu.async_copy(rows[b], acc.at[dst_v.at[jl]],
                                 sems[b], add=True)
            return carry

        lax.fori_loop(0, pb // 2, step, 0)
        pltpu.make_async_copy(rows[1], acc.at[dst_v.at[(pb - 1) % CH]],
                              sems[1]).wait()
        plsc.subcore_barrier()
        r0 = jnp.minimum(s * CPT, N - CPT)
        pltpu.sync_copy(acc.at[pl.ds(r0, CPT)],
                        out_hbm.at[c, pl.ds(r0, CPT)])

    return agg_kernel


def _mm_body(x_ref, w_ref, h_ref):
    h_ref[...] = jnp.dot(x_ref[...], w_ref[...],
                         preferred_element_type=jnp.float32)


def _mm(x, W1):
    return pl.pallas_call(
        _mm_body,
        grid=(GRID,),
        in_specs=[
            pl.BlockSpec((RB, D), lambda i: (i, 0)),
            pl.BlockSpec((D, D), lambda i: (0, 0)),
        ],
        out_specs=pl.BlockSpec((RB, D), lambda i: (i, 0)),
        out_shape=jax.ShapeDtypeStruct((N, D), jnp.float32),
    )(x, W1)


def _scale_body(h_ref, d0_ref, d1_ref, hs_ref, dis_ref):
    dis = lax.rsqrt(d0_ref[...] + d1_ref[...] + 1.0)
    hs_ref[...] = h_ref[...] * dis
    dis_ref[...] = dis


def _scale(h, d0, d1):
    return pl.pallas_call(
        _scale_body,
        grid=(GRID,),
        in_specs=[
            pl.BlockSpec((RB, D), lambda i: (i, 0)),
            pl.BlockSpec((RB, 1), lambda i: (i, 0)),
            pl.BlockSpec((RB, 1), lambda i: (i, 0)),
        ],
        out_specs=[
            pl.BlockSpec((RB, D), lambda i: (i, 0)),
            pl.BlockSpec((RB, 1), lambda i: (i, 0)),
        ],
        out_shape=[
            jax.ShapeDtypeStruct((N, D), jnp.float32),
            jax.ShapeDtypeStruct((N, 1), jnp.float32),
        ],
    )(h, d0, d1)


def _mid_body(a0_ref, a1_ref, hp_ref, dis_ref, b_ref, w_ref, hs_ref):
    dis = dis_ref[...]
    t = dis * (a0_ref[0] + a1_ref[0] + hp_ref[...]) + b_ref[...]
    t = jnp.where(t > 0.0, t, jnp.exp(t) - 1.0)
    hs_ref[...] = jnp.dot(
        t, w_ref[...], preferred_element_type=jnp.float32) * dis


def _mid(a, hs_prev, dis, bias, W):
    return pl.pallas_call(
        _mid_body,
        grid=(GRID,),
        in_specs=[
            pl.BlockSpec((1, RB, D), lambda i: (0, i, 0)),
            pl.BlockSpec((1, RB, D), lambda i: (1, i, 0)),
            pl.BlockSpec((RB, D), lambda i: (i, 0)),
            pl.BlockSpec((RB, 1), lambda i: (i, 0)),
            pl.BlockSpec((1, D), lambda i: (0, 0)),
            pl.BlockSpec((D, D), lambda i: (0, 0)),
        ],
        out_specs=pl.BlockSpec((RB, D), lambda i: (i, 0)),
        out_shape=jax.ShapeDtypeStruct((N, D), jnp.float32),
    )(a, a, hs_prev, dis, bias, W)


def _fin_body(a0_ref, a1_ref, hp_ref, dis_ref, b_ref, o_ref):
    o_ref[...] = (dis_ref[...] * (a0_ref[0] + a1_ref[0] + hp_ref[...])
                  + b_ref[...])


def _fin(a, hs_prev, dis, bias):
    return pl.pallas_call(
        _fin_body,
        grid=(GRID,),
        in_specs=[
            pl.BlockSpec((1, RB, D), lambda i: (0, i, 0)),
            pl.BlockSpec((1, RB, D), lambda i: (1, i, 0)),
            pl.BlockSpec((RB, D), lambda i: (i, 0)),
            pl.BlockSpec((RB, 1), lambda i: (i, 0)),
            pl.BlockSpec((1, D), lambda i: (0, 0)),
        ],
        out_specs=pl.BlockSpec((RB, D), lambda i: (i, 0)),
        out_shape=jax.ShapeDtypeStruct((N, D), jnp.float32),
    )(a, a, hs_prev, dis, bias)


def kernel(x, edge_index, W1, b1, W2, b2, W3, b3):
    e = edge_index.shape[1]
    pb = _pb(e)
    epad = NW * B * pb
    npad = epad - e

    pad_ar = lax.iota(jnp.int32, npad)
    srcp = jnp.concatenate([edge_index[0], pad_ar % N])
    dstp = jnp.concatenate([edge_index[1], N + (pad_ar % 16)])
    pbs = -(-pb // CH) * CH
    src3 = jnp.pad(srcp.reshape(NW, pb, B), ((0, 0), (0, pbs - pb), (0, 0)))
    dpad = N + (lax.iota(jnp.int32, B) % 16)
    dst3 = jnp.concatenate(
        [dstp.reshape(NW, pb, B),
         jnp.broadcast_to(dpad, (NW, pbs - pb, B))], axis=1)

    zcol = jnp.zeros((ACC_ROWS,), jnp.float32)
    zrows = jnp.zeros((RPT, D), jnp.float32)

    h1 = _mm(x, W1)
    degp = _make_deg_kernel(pb, pbs)(dst3, zcol)
    d0 = degp[0, :N, None]
    d1 = degp[1, :N, None]

    agg = _make_agg_kernel(pb)
    hs1, dis = _scale(h1, d0, d1)
    a1 = agg(hs1, src3, dst3, zrows)
    hs2 = _mid(a1, hs1, dis, b1.reshape(1, D), W2)
    a2 = agg(hs2, src3, dst3, zrows)
    hs3 = _mid(a2, hs2, dis, b2.reshape(1, D), W3)
    a3 = agg(hs3, src3, dst3, zrows)
    return _fin(a3, hs3, dis, b3.reshape(1, D))


# speedup vs baseline: 1.0928x; 1.0249x over previous
import functools

import jax
import jax.numpy as jnp
from jax import lax
from jax.experimental import pallas as pl
from jax.experimental.pallas import tpu as pltpu
from jax.experimental.pallas import tpu_sc as plsc

N = 10000
D = 128
NC = 2
NS = 16
NW = NC * NS
B = 128
CH = 48
RPT = 640
ACC_ROWS = NS * RPT
CPT = 632
RB = 2000
GRID = N // RB

_mesh = plsc.VectorSubcoreMesh(core_axis_name="c", subcore_axis_name="s")


def _pb(num_edges):
    pb = -(-num_edges // (NW * B))
    return pb + (pb % 2)


def _make_deg_kernel(pb, pbs):
    @functools.partial(
        pl.kernel,
        mesh=_mesh,
        out_type=jax.ShapeDtypeStruct((NC, ACC_ROWS), jnp.float32),
        scratch_types=[
            pltpu.VMEM((pbs, B), jnp.int32),
            pltpu.VMEM((B,), jnp.float32),
            pltpu.VMEM_SHARED((ACC_ROWS,), jnp.float32),
        ],
    )
    def deg_kernel(dst_hbm, zeros_hbm, out_hbm, dst_v, ones_v, acc):
        c = lax.axis_index("c")
        s = lax.axis_index("s")
        w = c * NS + s
        pltpu.sync_copy(dst_hbm.at[w], dst_v)
        pltpu.sync_copy(zeros_hbm.at[pl.ds(s * RPT, RPT)],
                        acc.at[pl.ds(s * RPT, RPT)])
        for k in range(B // 16):
            ones_v[pl.ds(16 * k, 16)] = jnp.ones((16,), jnp.float32)
        plsc.subcore_barrier()

        def step(j, carry):
            pltpu.sync_copy(ones_v, acc.at[dst_v.at[j]], add=True)
            return carry

        lax.fori_loop(0, pb, step, 0)
        plsc.subcore_barrier()
        pltpu.sync_copy(acc.at[pl.ds(s * RPT, RPT)],
                        out_hbm.at[c, pl.ds(s * RPT, RPT)])

    return deg_kernel


def _make_agg_kernel(pb):
    @functools.partial(
        pl.kernel,
        mesh=_mesh,
        out_type=jax.ShapeDtypeStruct((NC, N, D), jnp.float32),
        scratch_types=[
            pltpu.VMEM((CH, B), jnp.int32),
            pltpu.VMEM((CH, B), jnp.int32),
            pltpu.VMEM((B, D), jnp.float32),
            pltpu.VMEM((B, D), jnp.float32),
            pltpu.VMEM_SHARED((ACC_ROWS, D), jnp.float32),
            pltpu.SemaphoreType.DMA,
            pltpu.SemaphoreType.DMA,
            pltpu.SemaphoreType.DMA,
            pltpu.SemaphoreType.DMA,
            pltpu.SemaphoreType.DMA,
        ],
    )
    def agg_kernel(hs_hbm, src_hbm, dst_hbm, zeros_hbm, out_hbm,
                   src_v, dst_v, rows0, rows1, acc,
                   semg0, semg1, sems0, sems1, semz):
        c = lax.axis_index("c")
        s = lax.axis_index("s")
        w = c * NS + s
        pltpu.async_copy(zeros_hbm, acc.at[pl.ds(s * RPT, RPT)], semz)
        pltpu.sync_copy(src_hbm.at[w, pl.ds(0, CH)], src_v)
        pltpu.sync_copy(dst_hbm.at[w, pl.ds(0, CH)], dst_v)
        pltpu.async_copy(hs_hbm.at[src_v.at[0]], rows0, semg0)
        pltpu.make_async_copy(zeros_hbm,
                              acc.at[pl.ds(s * RPT, RPT)], semz).wait()
        plsc.subcore_barrier()

        rows = (rows0, rows1)
        semg = (semg0, semg1)
        sems = (sems0, sems1)

        def step(i, carry):
            for b in range(2):
                j = 2 * i + b
                jl = j % CH

                @pl.when(j > 0)
                def _():
                    pltpu.make_async_copy(rows[1 - b],
                                          acc.at[dst_v.at[(j - 1) % CH]],
                                          sems[1 - b]).wait()

                @pl.when(jnp.logical_and(jl == 0, j > 0))
                def _():
                    k0 = (j // CH) * CH
                    pltpu.sync_copy(src_hbm.at[w, pl.ds(k0, CH)], src_v)
                    pltpu.sync_copy(dst_hbm.at[w, pl.ds(k0, CH)], dst_v)
                    pltpu.async_copy(hs_hbm.at[src_v.at[0]],
                                     rows[b], semg[b])

                @pl.when(jnp.logical_and(j + 1 < pb, jl + 1 < CH))
                def _():
                    pltpu.async_copy(hs_hbm.at[src_v.at[jl + 1]],
                                     rows[1 - b], semg[1 - b])

                pltpu.make_async_copy(hs_hbm.at[src_v.at[jl]],
                                      rows[b], semg[b]).wait()
                pltpu.async_copy(rows[b], acc.at[dst_v.at[jl]],
                                 sems[b], add=True)
            return carry

        lax.fori_loop(0, pb // 2, step, 0)
        pltpu.make_async_copy(rows[1], acc.at[dst_v.at[(pb - 1) % CH]],
                              sems[1]).wait()
        plsc.subcore_barrier()
        r0 = jnp.minimum(s * CPT, N - CPT)
        pltpu.sync_copy(acc.at[pl.ds(r0, CPT)],
                        out_hbm.at[c, pl.ds(r0, CPT)])

    return agg_kernel


def _mm_body(x_ref, w_ref, h_ref):
    h_ref[...] = jnp.dot(x_ref[...], w_ref[...],
                         preferred_element_type=jnp.float32)


def _mm(x, W1):
    return pl.pallas_call(
        _mm_body,
        grid=(GRID,),
        in_specs=[
            pl.BlockSpec((RB, D), lambda i: (i, 0)),
            pl.BlockSpec((D, D), lambda i: (0, 0)),
        ],
        out_specs=pl.BlockSpec((RB, D), lambda i: (i, 0)),
        out_shape=jax.ShapeDtypeStruct((N, D), jnp.float32),
    )(x, W1)


def _scale_body(h_ref, d_ref, hs_ref, dis_ref):
    dis = lax.rsqrt(d_ref[...] + 1.0)
    hs_ref[...] = h_ref[...] * dis.reshape(RB, 1)
    dis_ref[...] = dis


def _scale(h, dsum):
    return pl.pallas_call(
        _scale_body,
        grid=(GRID,),
        in_specs=[
            pl.BlockSpec((RB, D), lambda i: (i, 0)),
            pl.BlockSpec((1, 1, RB), lambda i: (i, 0, 0)),
        ],
        out_specs=[
            pl.BlockSpec((RB, D), lambda i: (i, 0)),
            pl.BlockSpec((1, 1, RB), lambda i: (i, 0, 0)),
        ],
        out_shape=[
            jax.ShapeDtypeStruct((N, D), jnp.float32),
            jax.ShapeDtypeStruct((GRID, 1, RB), jnp.float32),
        ],
    )(h, dsum)


def _mid_body(a0_ref, a1_ref, hp_ref, dis_ref, b_ref, w_ref, hs_ref):
    dis = dis_ref[...].reshape(RB, 1)
    t = dis * (a0_ref[0] + a1_ref[0] + hp_ref[...]) + b_ref[...]
    t = jnp.where(t > 0.0, t, jnp.exp(t) - 1.0)
    hs_ref[...] = jnp.dot(
        t, w_ref[...], preferred_element_type=jnp.float32) * dis


def _mid(a, hs_prev, dis, bias, W):
    return pl.pallas_call(
        _mid_body,
        grid=(GRID,),
        in_specs=[
            pl.BlockSpec((1, RB, D), lambda i: (0, i, 0)),
            pl.BlockSpec((1, RB, D), lambda i: (1, i, 0)),
            pl.BlockSpec((RB, D), lambda i: (i, 0)),
            pl.BlockSpec((1, 1, RB), lambda i: (i, 0, 0)),
            pl.BlockSpec((1, D), lambda i: (0, 0)),
            pl.BlockSpec((D, D), lambda i: (0, 0)),
        ],
        out_specs=pl.BlockSpec((RB, D), lambda i: (i, 0)),
        out_shape=jax.ShapeDtypeStruct((N, D), jnp.float32),
    )(a, a, hs_prev, dis, bias, W)


def _fin_body(a0_ref, a1_ref, hp_ref, dis_ref, b_ref, o_ref):
    dis = dis_ref[...].reshape(RB, 1)
    o_ref[...] = (dis * (a0_ref[0] + a1_ref[0] + hp_ref[...])
                  + b_ref[...])


def _fin(a, hs_prev, dis, bias):
    return pl.pallas_call(
        _fin_body,
        grid=(GRID,),
        in_specs=[
            pl.BlockSpec((1, RB, D), lambda i: (0, i, 0)),
            pl.BlockSpec((1, RB, D), lambda i: (1, i, 0)),
            pl.BlockSpec((RB, D), lambda i: (i, 0)),
            pl.BlockSpec((1, 1, RB), lambda i: (i, 0, 0)),
            pl.BlockSpec((1, D), lambda i: (0, 0)),
        ],
        out_specs=pl.BlockSpec((RB, D), lambda i: (i, 0)),
        out_shape=jax.ShapeDtypeStruct((N, D), jnp.float32),
    )(a, a, hs_prev, dis, bias)


def kernel(x, edge_index, W1, b1, W2, b2, W3, b3):
    e = edge_index.shape[1]
    pb = _pb(e)
    epad = NW * B * pb
    npad = epad - e

    pad_ar = lax.iota(jnp.int32, npad)
    srcp = jnp.concatenate([edge_index[0], pad_ar % N])
    dstp = jnp.concatenate([edge_index[1], N + (pad_ar % 16)])
    pbs = -(-pb // CH) * CH
    src3 = jnp.pad(srcp.reshape(NW, pb, B), ((0, 0), (0, pbs - pb), (0, 0)))
    dpad = N + (lax.iota(jnp.int32, B) % 16)
    dst3 = jnp.concatenate(
        [dstp.reshape(NW, pb, B),
         jnp.broadcast_to(dpad, (NW, pbs - pb, B))], axis=1)

    zcol = jnp.zeros((ACC_ROWS,), jnp.float32)
    zrows = jnp.zeros((RPT, D), jnp.float32)

    h1 = _mm(x, W1)
    degp = _make_deg_kernel(pb, pbs)(dst3, zcol)
    dsum = (degp[0, :N] + degp[1, :N]).reshape(GRID, 1, RB)

    agg = _make_agg_kernel(pb)
    hs1, dis = _scale(h1, dsum)
    a1 = agg(hs1, src3, dst3, zrows)
    hs2 = _mid(a1, hs1, dis, b1.reshape(1, D), W2)
    a2 = agg(hs2, src3, dst3, zrows)
    hs3 = _mid(a2, hs2, dis, b2.reshape(1, D), W3)
    a3 = agg(hs3, src3, dst3, zrows)
    return _fin(a3, hs3, dis, b3.reshape(1, D))
